# baseline (speedup 1.0000x reference)
import functools

import jax
import jax.numpy as jnp
from jax import lax
from jax.experimental import pallas as pl
from jax.experimental.pallas import tpu as pltpu
from jax.experimental.pallas import tpu_sc as plsc

NUM_CORES = 2
NUM_SUBCORES = 16
NUM_WORKERS = NUM_CORES * NUM_SUBCORES

BATCH = 16384
EMBED = 128
RANK = EMBED // 2
LANES = 16

ROWS_PER_WORKER = BATCH // NUM_WORKERS
CHUNK = 128
NUM_CHUNKS = ROWS_PER_WORKER // CHUNK
NBUF = 2


def _score_body(s_hbm, p_hbm, o_hbm, ent_hbm, rel_hbm, out_hbm,
                sidx, pidx, oidx, se_bufs, pe_bufs, oe_bufs, out_v,
                idx_sems, gather_sems):
    sid = lax.axis_index("s")
    wid = sid * NUM_CORES + lax.axis_index("c")
    base = wid * ROWS_PER_WORKER

    idx_copies = []
    for c in range(NUM_CHUNKS):
        off = base + c * CHUNK
        sem = idx_sems[c]
        idx_copies.append((
            pltpu.async_copy(s_hbm.at[pl.ds(off, CHUNK)], sidx.at[c], sem),
            pltpu.async_copy(p_hbm.at[pl.ds(off, CHUNK)], pidx.at[c], sem),
            pltpu.async_copy(o_hbm.at[pl.ds(off, CHUNK)], oidx.at[c], sem),
        ))

    def fire(c):
        for cp in idx_copies[c]:
            cp.wait()
        b = c % NBUF
        sem = gather_sems[b]
        return (
            pltpu.async_copy(ent_hbm.at[sidx.at[c]], se_bufs[b], sem),
            pltpu.async_copy(rel_hbm.at[pidx.at[c]], pe_bufs[b], sem),
            pltpu.async_copy(ent_hbm.at[oidx.at[c]], oe_bufs[b], sem),
        )

    lanes = lax.iota(jnp.int32, LANES)
    in_flight = [None] * NBUF
    for c0 in range(NBUF - 1):
        in_flight[c0] = fire(c0)

    for c in range(NUM_CHUNKS):
        nxt = c + NBUF - 1
        if nxt < NUM_CHUNKS:
            in_flight[nxt % NBUF] = fire(nxt)
        for cp in in_flight[c % NBUF]:
            cp.wait()
        se_v = se_bufs[c % NBUF]
        pe_v = pe_bufs[c % NBUF]
        oe_v = oe_bufs[c % NBUF]

        UNROLL = 8

        def step_body(i, svec, c=c, se_v=se_v, pe_v=pe_v, oe_v=oe_v):
            kb = (i * UNROLL) % LANES
            for u in range(UNROLL):
                r = i * UNROLL + u
                acc = jnp.zeros((LANES,), jnp.float32)
                for j in range(RANK // LANES):
                    lo = j * LANES
                    hi = RANK + j * LANES
                    s_re = se_v[r, pl.ds(lo, LANES)]
                    s_im = se_v[r, pl.ds(hi, LANES)]
                    p_re = pe_v[r, pl.ds(lo, LANES)]
                    p_im = pe_v[r, pl.ds(hi, LANES)]
                    o_re = oe_v[r, pl.ds(lo, LANES)]
                    o_im = oe_v[r, pl.ds(hi, LANES)]
                    sp_re = s_re * p_re - s_im * p_im
                    sp_im = s_re * p_im + s_im * p_re
                    acc = acc + (sp_re * o_re + sp_im * o_im)
                for sh in (8, 4, 2, 1):
                    acc = acc + acc.at[lanes ^ sh].get(
                        mode="promise_in_bounds")
                svec = jnp.where(lanes == kb + u, acc, svec)
            group_full = (i % (LANES // UNROLL)) == (LANES // UNROLL - 1)

            @pl.when(group_full)
            def _():
                g = (i * UNROLL) // LANES
                out_v[pl.ds(c * CHUNK + g * LANES, LANES)] = svec

            return jnp.where(group_full, jnp.zeros((LANES,), jnp.float32),
                             svec)

        lax.fori_loop(0, CHUNK // UNROLL, step_body,
                      jnp.zeros((LANES,), jnp.float32))

    pltpu.sync_copy(out_v, out_hbm.at[pl.ds(base, ROWS_PER_WORKER)])


@functools.partial(
    pl.kernel,
    out_type=jax.ShapeDtypeStruct((BATCH,), jnp.float32),
    mesh=plsc.VectorSubcoreMesh(core_axis_name="c", subcore_axis_name="s"),
    scratch_types=[
        pltpu.VMEM((NUM_CHUNKS, CHUNK), jnp.int32),
        pltpu.VMEM((NUM_CHUNKS, CHUNK), jnp.int32),
        pltpu.VMEM((NUM_CHUNKS, CHUNK), jnp.int32),
        [pltpu.VMEM((CHUNK, EMBED), jnp.float32)] * NBUF,
        [pltpu.VMEM((CHUNK, EMBED), jnp.float32)] * NBUF,
        [pltpu.VMEM((CHUNK, EMBED), jnp.float32)] * NBUF,
        pltpu.VMEM((ROWS_PER_WORKER,), jnp.float32),
        [pltpu.SemaphoreType.DMA] * NUM_CHUNKS,
        [pltpu.SemaphoreType.DMA] * NBUF,
    ],
)
def _complex_score_sc(s_hbm, p_hbm, o_hbm, ent_hbm, rel_hbm, out_hbm,
                      sidx, pidx, oidx, se_bufs, pe_bufs, oe_bufs, out_v,
                      idx_sems, gather_sems):
    _score_body(s_hbm, p_hbm, o_hbm, ent_hbm, rel_hbm, out_hbm,
                sidx, pidx, oidx, se_bufs, pe_bufs, oe_bufs, out_v,
                idx_sems, gather_sems)


def kernel(s, p, o, entity_emb, relation_emb):
    out = _complex_score_sc(s.astype(jnp.int32), p.astype(jnp.int32),
                            o.astype(jnp.int32), entity_emb, relation_emb)
    return out.reshape(BATCH, 1)


# speedup vs baseline: 1.1298x; 1.1298x over previous
import functools

import jax
import jax.numpy as jnp
from jax import lax
from jax.experimental import pallas as pl
from jax.experimental.pallas import tpu as pltpu
from jax.experimental.pallas import tpu_sc as plsc

NUM_CORES = 2
NUM_SUBCORES = 16
NUM_WORKERS = NUM_CORES * NUM_SUBCORES

BATCH = 16384
EMBED = 128
RANK = EMBED // 2
LANES = 16

ROWS_PER_WORKER = BATCH // NUM_WORKERS
CHUNK = 128
NUM_CHUNKS = ROWS_PER_WORKER // CHUNK
NBUF = 2


def _score_body(s_hbm, p_hbm, o_hbm, ent_hbm, rel_hbm, out_hbm,
                sidx, pidx, oidx, se_bufs, pe_bufs, oe_bufs, out_v,
                idx_sems, gather_sems):
    sid = lax.axis_index("s")
    wid = sid * NUM_CORES + lax.axis_index("c")
    base = wid * ROWS_PER_WORKER

    idx_copies = []
    for c in range(NUM_CHUNKS):
        off = base + c * CHUNK
        sem = idx_sems[c]
        idx_copies.append((
            pltpu.async_copy(s_hbm.at[pl.ds(off, CHUNK)], sidx.at[c], sem),
            pltpu.async_copy(p_hbm.at[pl.ds(off, CHUNK)], pidx.at[c], sem),
            pltpu.async_copy(o_hbm.at[pl.ds(off, CHUNK)], oidx.at[c], sem),
        ))

    def fire(c):
        for cp in idx_copies[c]:
            cp.wait()
        b = c % NBUF
        sem = gather_sems[b]
        return (
            pltpu.async_copy(ent_hbm.at[sidx.at[c]], se_bufs[b], sem),
            pltpu.async_copy(rel_hbm.at[pidx.at[c]], pe_bufs[b], sem),
            pltpu.async_copy(ent_hbm.at[oidx.at[c]], oe_bufs[b], sem),
        )

    lanes = lax.iota(jnp.int32, LANES)
    in_flight = [None] * NBUF
    for c0 in range(NBUF - 1):
        in_flight[c0] = fire(c0)

    for c in range(NUM_CHUNKS):
        nxt = c + NBUF - 1
        if nxt < NUM_CHUNKS:
            in_flight[nxt % NBUF] = fire(nxt)
        for cp in in_flight[c % NBUF]:
            cp.wait()
        se_v = se_bufs[c % NBUF]
        pe_v = pe_bufs[c % NBUF]
        oe_v = oe_bufs[c % NBUF]

        UNROLL = 2

        def step_body(i, svec, c=c, se_v=se_v, pe_v=pe_v, oe_v=oe_v):
            kb = (i * UNROLL) % LANES
            for u in range(UNROLL):
                r = i * UNROLL + u
                acc = jnp.zeros((LANES,), jnp.float32)
                for j in range(RANK // LANES):
                    lo = j * LANES
                    hi = RANK + j * LANES
                    s_re = se_v[r, pl.ds(lo, LANES)]
                    s_im = se_v[r, pl.ds(hi, LANES)]
                    p_re = pe_v[r, pl.ds(lo, LANES)]
                    p_im = pe_v[r, pl.ds(hi, LANES)]
                    o_re = oe_v[r, pl.ds(lo, LANES)]
                    o_im = oe_v[r, pl.ds(hi, LANES)]
                    sp_re = s_re * p_re - s_im * p_im
                    sp_im = s_re * p_im + s_im * p_re
                    acc = acc + (sp_re * o_re + sp_im * o_im)
                for sh in (8, 4, 2, 1):
                    acc = acc + acc.at[lanes ^ sh].get(
                        mode="promise_in_bounds")
                svec = jnp.where(lanes == kb + u, acc, svec)
            group_full = (i % (LANES // UNROLL)) == (LANES // UNROLL - 1)

            @pl.when(group_full)
            def _():
                g = (i * UNROLL) // LANES
                out_v[pl.ds(c * CHUNK + g * LANES, LANES)] = svec

            return jnp.where(group_full, jnp.zeros((LANES,), jnp.float32),
                             svec)

        lax.fori_loop(0, CHUNK // UNROLL, step_body,
                      jnp.zeros((LANES,), jnp.float32))

    pltpu.sync_copy(out_v, out_hbm.at[pl.ds(base, ROWS_PER_WORKER)])


@functools.partial(
    pl.kernel,
    out_type=jax.ShapeDtypeStruct((BATCH,), jnp.float32),
    mesh=plsc.VectorSubcoreMesh(core_axis_name="c", subcore_axis_name="s"),
    scratch_types=[
        pltpu.VMEM((NUM_CHUNKS, CHUNK), jnp.int32),
        pltpu.VMEM((NUM_CHUNKS, CHUNK), jnp.int32),
        pltpu.VMEM((NUM_CHUNKS, CHUNK), jnp.int32),
        [pltpu.VMEM((CHUNK, EMBED), jnp.float32)] * NBUF,
        [pltpu.VMEM((CHUNK, EMBED), jnp.float32)] * NBUF,
        [pltpu.VMEM((CHUNK, EMBED), jnp.float32)] * NBUF,
        pltpu.VMEM((ROWS_PER_WORKER,), jnp.float32),
        [pltpu.SemaphoreType.DMA] * NUM_CHUNKS,
        [pltpu.SemaphoreType.DMA] * NBUF,
    ],
)
def _complex_score_sc(s_hbm, p_hbm, o_hbm, ent_hbm, rel_hbm, out_hbm,
                      sidx, pidx, oidx, se_bufs, pe_bufs, oe_bufs, out_v,
                      idx_sems, gather_sems):
    _score_body(s_hbm, p_hbm, o_hbm, ent_hbm, rel_hbm, out_hbm,
                sidx, pidx, oidx, se_bufs, pe_bufs, oe_bufs, out_v,
                idx_sems, gather_sems)


def kernel(s, p, o, entity_emb, relation_emb):
    out = _complex_score_sc(s.astype(jnp.int32), p.astype(jnp.int32),
                            o.astype(jnp.int32), entity_emb, relation_emb)
    return out.reshape(BATCH, 1)


# speedup vs baseline: 1.1346x; 1.0042x over previous
import functools

import jax
import jax.numpy as jnp
from jax import lax
from jax.experimental import pallas as pl
from jax.experimental.pallas import tpu as pltpu
from jax.experimental.pallas import tpu_sc as plsc

NUM_CORES = 2
NUM_SUBCORES = 16
NUM_WORKERS = NUM_CORES * NUM_SUBCORES

BATCH = 16384
EMBED = 128
RANK = EMBED // 2
LANES = 16

ROWS_PER_WORKER = BATCH // NUM_WORKERS
CHUNK = 128
NUM_CHUNKS = ROWS_PER_WORKER // CHUNK
NBUF = 2


def _score_body(s_hbm, p_hbm, o_hbm, ent_hbm, rel_hbm, out_hbm,
                sidx, pidx, oidx, se_bufs, pe_bufs, oe_bufs, out_v,
                idx_sems, gather_sems):
    sid = lax.axis_index("s")
    wid = sid * NUM_CORES + lax.axis_index("c")
    base = wid * ROWS_PER_WORKER

    idx_copies = []
    for c in range(NUM_CHUNKS):
        off = base + c * CHUNK
        sem = idx_sems[c]
        idx_copies.append((
            pltpu.async_copy(s_hbm.at[pl.ds(off, CHUNK)], sidx.at[c], sem),
            pltpu.async_copy(p_hbm.at[pl.ds(off, CHUNK)], pidx.at[c], sem),
            pltpu.async_copy(o_hbm.at[pl.ds(off, CHUNK)], oidx.at[c], sem),
        ))

    def fire(c):
        for cp in idx_copies[c]:
            cp.wait()
        b = c % NBUF
        sem = gather_sems[b]
        return (
            pltpu.async_copy(ent_hbm.at[sidx.at[c]], se_bufs[b], sem),
            pltpu.async_copy(rel_hbm.at[pidx.at[c]], pe_bufs[b], sem),
            pltpu.async_copy(ent_hbm.at[oidx.at[c]], oe_bufs[b], sem),
        )

    lanes = lax.iota(jnp.int32, LANES)
    in_flight = [None] * NBUF
    for c0 in range(NBUF - 1):
        in_flight[c0] = fire(c0)

    for c in range(NUM_CHUNKS):
        nxt = c + NBUF - 1
        if nxt < NUM_CHUNKS:
            in_flight[nxt % NBUF] = fire(nxt)
        for cp in in_flight[c % NBUF]:
            cp.wait()
        se_v = se_bufs[c % NBUF]
        pe_v = pe_bufs[c % NBUF]
        oe_v = oe_bufs[c % NBUF]

        UNROLL = 4

        def step_body(i, svec, c=c, se_v=se_v, pe_v=pe_v, oe_v=oe_v):
            kb = (i * UNROLL) % LANES
            for u in range(UNROLL):
                r = i * UNROLL + u
                acc = jnp.zeros((LANES,), jnp.float32)
                for j in range(RANK // LANES):
                    lo = j * LANES
                    hi = RANK + j * LANES
                    s_re = se_v[r, pl.ds(lo, LANES)]
                    s_im = se_v[r, pl.ds(hi, LANES)]
                    p_re = pe_v[r, pl.ds(lo, LANES)]
                    p_im = pe_v[r, pl.ds(hi, LANES)]
                    o_re = oe_v[r, pl.ds(lo, LANES)]
                    o_im = oe_v[r, pl.ds(hi, LANES)]
                    sp_re = s_re * p_re - s_im * p_im
                    sp_im = s_re * p_im + s_im * p_re
                    acc = acc + (sp_re * o_re + sp_im * o_im)
                for sh in (8, 4, 2, 1):
                    acc = acc + acc.at[lanes ^ sh].get(
                        mode="promise_in_bounds")
                svec = jnp.where(lanes == kb + u, acc, svec)
            group_full = (i % (LANES // UNROLL)) == (LANES // UNROLL - 1)

            @pl.when(group_full)
            def _():
                g = (i * UNROLL) // LANES
                out_v[pl.ds(c * CHUNK + g * LANES, LANES)] = svec

            return jnp.where(group_full, jnp.zeros((LANES,), jnp.float32),
                             svec)

        lax.fori_loop(0, CHUNK // UNROLL, step_body,
                      jnp.zeros((LANES,), jnp.float32))

    pltpu.sync_copy(out_v, out_hbm.at[pl.ds(base, ROWS_PER_WORKER)])


@functools.partial(
    pl.kernel,
    out_type=jax.ShapeDtypeStruct((BATCH,), jnp.float32),
    mesh=plsc.VectorSubcoreMesh(core_axis_name="c", subcore_axis_name="s"),
    scratch_types=[
        pltpu.VMEM((NUM_CHUNKS, CHUNK), jnp.int32),
        pltpu.VMEM((NUM_CHUNKS, CHUNK), jnp.int32),
        pltpu.VMEM((NUM_CHUNKS, CHUNK), jnp.int32),
        [pltpu.VMEM((CHUNK, EMBED), jnp.float32)] * NBUF,
        [pltpu.VMEM((CHUNK, EMBED), jnp.float32)] * NBUF,
        [pltpu.VMEM((CHUNK, EMBED), jnp.float32)] * NBUF,
        pltpu.VMEM((ROWS_PER_WORKER,), jnp.float32),
        [pltpu.SemaphoreType.DMA] * NUM_CHUNKS,
        [pltpu.SemaphoreType.DMA] * NBUF,
    ],
)
def _complex_score_sc(s_hbm, p_hbm, o_hbm, ent_hbm, rel_hbm, out_hbm,
                      sidx, pidx, oidx, se_bufs, pe_bufs, oe_bufs, out_v,
                      idx_sems, gather_sems):
    _score_body(s_hbm, p_hbm, o_hbm, ent_hbm, rel_hbm, out_hbm,
                sidx, pidx, oidx, se_bufs, pe_bufs, oe_bufs, out_v,
                idx_sems, gather_sems)


def kernel(s, p, o, entity_emb, relation_emb):
    out = _complex_score_sc(s.astype(jnp.int32), p.astype(jnp.int32),
                            o.astype(jnp.int32), entity_emb, relation_emb)
    return out.reshape(BATCH, 1)


# speedup vs baseline: 1.1943x; 1.0527x over previous
import functools

import jax
import jax.numpy as jnp
from jax import lax
from jax.experimental import pallas as pl
from jax.experimental.pallas import tpu as pltpu
from jax.experimental.pallas import tpu_sc as plsc

NUM_CORES = 2
NUM_SUBCORES = 16
NUM_WORKERS = NUM_CORES * NUM_SUBCORES

BATCH = 16384
EMBED = 128
RANK = EMBED // 2
LANES = 16

ROWS_PER_WORKER = BATCH // NUM_WORKERS
CHUNK = 128
CHUNK_SIZES = (64, 128, 128, 128, 64)
CHUNK_OFFS = (0, 64, 192, 320, 448)
NUM_CHUNKS = len(CHUNK_SIZES)
NBUF = 2


def _score_body(s_hbm, p_hbm, o_hbm, ent_hbm, rel_hbm, out_hbm,
                sidx, pidx, oidx, se_bufs, pe_bufs, oe_bufs, out_v,
                idx_sems, gather_sems):
    sid = lax.axis_index("s")
    wid = sid * NUM_CORES + lax.axis_index("c")
    base = wid * ROWS_PER_WORKER

    idx_copies = []
    for c in range(NUM_CHUNKS):
        off = base + CHUNK_OFFS[c]
        n = CHUNK_SIZES[c]
        sem = idx_sems[c]
        idx_copies.append((
            pltpu.async_copy(s_hbm.at[pl.ds(off, n)],
                             sidx.at[c, pl.ds(0, n)], sem),
            pltpu.async_copy(p_hbm.at[pl.ds(off, n)],
                             pidx.at[c, pl.ds(0, n)], sem),
            pltpu.async_copy(o_hbm.at[pl.ds(off, n)],
                             oidx.at[c, pl.ds(0, n)], sem),
        ))

    def fire(c):
        for cp in idx_copies[c]:
            cp.wait()
        b = c % NBUF
        n = CHUNK_SIZES[c]
        sem = gather_sems[b]
        return (
            pltpu.async_copy(ent_hbm.at[sidx.at[c, pl.ds(0, n)]],
                             se_bufs[b].at[pl.ds(0, n)], sem),
            pltpu.async_copy(rel_hbm.at[pidx.at[c, pl.ds(0, n)]],
                             pe_bufs[b].at[pl.ds(0, n)], sem),
            pltpu.async_copy(ent_hbm.at[oidx.at[c, pl.ds(0, n)]],
                             oe_bufs[b].at[pl.ds(0, n)], sem),
        )

    lanes = lax.iota(jnp.int32, LANES)
    in_flight = [None] * NBUF
    for c0 in range(NBUF - 1):
        in_flight[c0] = fire(c0)

    for c in range(NUM_CHUNKS):
        nxt = c + NBUF - 1
        if nxt < NUM_CHUNKS:
            in_flight[nxt % NBUF] = fire(nxt)
        for cp in in_flight[c % NBUF]:
            cp.wait()
        se_v = se_bufs[c % NBUF]
        pe_v = pe_bufs[c % NBUF]
        oe_v = oe_bufs[c % NBUF]

        UNROLL = 4

        def step_body(i, svec, c=c, se_v=se_v, pe_v=pe_v, oe_v=oe_v):
            kb = (i * UNROLL) % LANES
            for u in range(UNROLL):
                r = i * UNROLL + u
                acc = jnp.zeros((LANES,), jnp.float32)
                for j in range(RANK // LANES):
                    lo = j * LANES
                    hi = RANK + j * LANES
                    s_re = se_v[r, pl.ds(lo, LANES)]
                    s_im = se_v[r, pl.ds(hi, LANES)]
                    p_re = pe_v[r, pl.ds(lo, LANES)]
                    p_im = pe_v[r, pl.ds(hi, LANES)]
                    o_re = oe_v[r, pl.ds(lo, LANES)]
                    o_im = oe_v[r, pl.ds(hi, LANES)]
                    sp_re = s_re * p_re - s_im * p_im
                    sp_im = s_re * p_im + s_im * p_re
                    acc = acc + (sp_re * o_re + sp_im * o_im)
                for sh in (8, 4, 2, 1):
                    acc = acc + acc.at[lanes ^ sh].get(
                        mode="promise_in_bounds")
                svec = jnp.where(lanes == kb + u, acc, svec)
            group_full = (i % (LANES // UNROLL)) == (LANES // UNROLL - 1)

            @pl.when(group_full)
            def _():
                g = (i * UNROLL) // LANES
                out_v[pl.ds(CHUNK_OFFS[c] + g * LANES, LANES)] = svec

            return jnp.where(group_full, jnp.zeros((LANES,), jnp.float32),
                             svec)

        lax.fori_loop(0, CHUNK_SIZES[c] // UNROLL, step_body,
                      jnp.zeros((LANES,), jnp.float32))

    pltpu.sync_copy(out_v, out_hbm.at[pl.ds(base, ROWS_PER_WORKER)])


@functools.partial(
    pl.kernel,
    out_type=jax.ShapeDtypeStruct((BATCH,), jnp.float32),
    mesh=plsc.VectorSubcoreMesh(core_axis_name="c", subcore_axis_name="s"),
    scratch_types=[
        pltpu.VMEM((NUM_CHUNKS, CHUNK), jnp.int32),
        pltpu.VMEM((NUM_CHUNKS, CHUNK), jnp.int32),
        pltpu.VMEM((NUM_CHUNKS, CHUNK), jnp.int32),
        [pltpu.VMEM((CHUNK, EMBED), jnp.float32)] * NBUF,
        [pltpu.VMEM((CHUNK, EMBED), jnp.float32)] * NBUF,
        [pltpu.VMEM((CHUNK, EMBED), jnp.float32)] * NBUF,
        pltpu.VMEM((ROWS_PER_WORKER,), jnp.float32),
        [pltpu.SemaphoreType.DMA] * NUM_CHUNKS,
        [pltpu.SemaphoreType.DMA] * NBUF,
    ],
)
def _complex_score_sc(s_hbm, p_hbm, o_hbm, ent_hbm, rel_hbm, out_hbm,
                      sidx, pidx, oidx, se_bufs, pe_bufs, oe_bufs, out_v,
                      idx_sems, gather_sems):
    _score_body(s_hbm, p_hbm, o_hbm, ent_hbm, rel_hbm, out_hbm,
                sidx, pidx, oidx, se_bufs, pe_bufs, oe_bufs, out_v,
                idx_sems, gather_sems)


def kernel(s, p, o, entity_emb, relation_emb):
    out = _complex_score_sc(s.astype(jnp.int32), p.astype(jnp.int32),
                            o.astype(jnp.int32), entity_emb, relation_emb)
    return out.reshape(BATCH, 1)
